# baseline (speedup 1.0000x reference)
import functools

import jax
import jax.numpy as jnp
from jax import lax
from jax.experimental import pallas as pl
from jax.experimental.pallas import tpu as pltpu
from jax.experimental.pallas import tpu_sc as plsc

D = 128
C = 128
NC = 2
NS = 16
L = 16
NBUF = 2
FRAC1 = 0.30


def _decode(z, src, dst, cpw0, cpw1):
    e_pad = src.shape[0]
    mesh = plsc.VectorSubcoreMesh(core_axis_name="c", subcore_axis_name="s")

    @functools.partial(
        pl.kernel,
        mesh=mesh,
        compiler_params=pltpu.CompilerParams(needs_layout_passes=False,
                                             use_tc_tiling_on_sc=False),
        out_type=jax.ShapeDtypeStruct((e_pad,), jnp.float32),
        scratch_types=(
            [pltpu.VMEM((C,), jnp.int32) for _ in range(2 * NBUF)]
            + [pltpu.VMEM((C, D // 2), jnp.int32) for _ in range(2 * NBUF)]
            + [pltpu.VMEM((C,), jnp.float32) for _ in range(NBUF)]
            + [pltpu.SemaphoreType.DMA for _ in range(3 * NBUF)]
        ),
    )
    def kern(z_hbm, src_hbm, dst_hbm, out_hbm,
             sidx0, sidx1, didx0, didx1, sr0, sr1, dr0, dr1, ov0, ov1,
             gs0, gs1, gd0, gd1, os0, os1):
        sidx = (sidx0, sidx1)
        didx = (didx0, didx1)
        srows = (sr0, sr1)
        drows = (dr0, dr1)
        outv = (ov0, ov1)
        gsem = (gs0, gs1)
        dsem = (gd0, gd1)
        osem = (os0, os1)

        c = lax.axis_index("c")
        s = lax.axis_index("s")
        cpw = jnp.where(c == 0, cpw0, cpw1)
        base_chunk = jnp.where(c == 0, s * cpw0, NS * cpw0 + s * cpw1)
        base0 = base_chunk * C
        lane = lax.broadcasted_iota(jnp.int32, (L,), 0)

        def stage(j, b):
            off = base0 + j * C
            pltpu.sync_copy(src_hbm.at[pl.ds(off, C)], sidx[b])
            pltpu.sync_copy(dst_hbm.at[pl.ds(off, C)], didx[b])
            pltpu.async_copy(z_hbm.at[sidx[b]], srows[b], gsem[b])
            pltpu.async_copy(z_hbm.at[didx[b]], drows[b], dsem[b])

        def compute(b):
            def group_body(g, carry2):
                def quad(q, res):
                    for i in range(4):
                        e = g * L + q * 4 + i
                        acc = jnp.zeros((L,), jnp.float32)
                        for k32 in range(D // (2 * L)):
                            a = plsc.bitcast(
                                srows[b][e, pl.ds(k32 * L, L)], jnp.bfloat16)
                            bb = plsc.bitcast(
                                drows[b][e, pl.ds(k32 * L, L)], jnp.bfloat16)
                            a0, a1 = plsc.unpack(
                                a, format=plsc.PackFormat.INTERLEAVED)
                            b0, b1 = plsc.unpack(
                                bb, format=plsc.PackFormat.INTERLEAVED)
                            acc = acc + a0 * b0
                            acc = acc + a1 * b1
                        res = jnp.where(lane == q * 4 + i, jnp.sum(acc), res)
                    return res

                res = lax.fori_loop(0, 4, quad, jnp.zeros((L,), jnp.float32))
                outv[b][pl.ds(g * L, L)] = res
                return carry2

            lax.fori_loop(0, C // L, group_body, 0)

        for b in range(NBUF):
            stage(b, b)

        def loop_body(i, carry):
            for b in range(NBUF):
                j = i * NBUF + b
                pltpu.make_async_copy(z_hbm.at[sidx[b]], srows[b],
                                      gsem[b]).wait()
                pltpu.make_async_copy(z_hbm.at[didx[b]], drows[b],
                                      dsem[b]).wait()

                @pl.when(j >= NBUF)
                def _():
                    pltpu.make_async_copy(outv[b],
                                          out_hbm.at[pl.ds(base0, C)],
                                          osem[b]).wait()

                compute(b)
                off = base0 + j * C
                pltpu.async_copy(outv[b], out_hbm.at[pl.ds(off, C)], osem[b])

                nj = j + NBUF

                @pl.when(nj < cpw)
                def _():
                    stage(nj, b)
            return carry

        lax.fori_loop(0, cpw // NBUF, loop_body, 0)

        for b in range(NBUF):
            pltpu.make_async_copy(outv[b], out_hbm.at[pl.ds(base0, C)],
                                  osem[b]).wait()

    return kern(z, src, dst)


def kernel(features, graph, pos_edge, neg_edge):
    zb = features[-1].astype(jnp.bfloat16)
    z = lax.bitcast_convert_type(zb.reshape(zb.shape[0], D // 2, 2),
                                 jnp.int32)
    edge = jnp.concatenate([pos_edge, neg_edge], axis=-1)
    e = edge.shape[1]
    unit = NS * C * NBUF
    t = -(-e // unit) * NBUF
    cpw1 = max(NBUF, int(t * FRAC1 / NBUF) * NBUF)
    cpw0 = t - cpw1
    e_pad = NS * t * C
    src = jnp.pad(edge[0], (0, e_pad - e))
    dst = jnp.pad(edge[1], (0, e_pad - e))
    out = _decode(z, src, dst, cpw0, cpw1)
    return out[:e]


# speedup vs baseline: 1.1800x; 1.1800x over previous
import functools

import jax
import jax.numpy as jnp
from jax import lax
from jax.experimental import pallas as pl
from jax.experimental.pallas import tpu as pltpu
from jax.experimental.pallas import tpu_sc as plsc

D = 128
C = 128
NC = 2
NS = 16
L = 16
NBUF = 2
SUP = 4
FRAC1 = 0.30


def _decode(z, src, dst, cpw0, cpw1):
    e_pad = src.shape[0]
    mesh = plsc.VectorSubcoreMesh(core_axis_name="c", subcore_axis_name="s")

    @functools.partial(
        pl.kernel,
        mesh=mesh,
        compiler_params=pltpu.CompilerParams(needs_layout_passes=False),
        out_type=jax.ShapeDtypeStruct((e_pad,), jnp.float32),
        scratch_types=(
            [pltpu.VMEM((SUP * C,), jnp.int32) for _ in range(4)]
            + [pltpu.VMEM((C, D), jnp.float32) for _ in range(2 * NBUF)]
            + [pltpu.VMEM((C,), jnp.float32) for _ in range(NBUF)]
            + [pltpu.SemaphoreType.DMA for _ in range(10)]
        ),
    )
    def kern(z_hbm, src_hbm, dst_hbm, out_hbm,
             sb0, sb1, db0, db1, sr0, sr1, dr0, dr1, ov0, ov1,
             bs0, bs1, bd0, bd1, gs0, gs1, gd0, gd1, os0, os1):
        sbatch = (sb0, sb1)
        dbatch = (db0, db1)
        srows = (sr0, sr1)
        drows = (dr0, dr1)
        outv = (ov0, ov1)
        bssem = (bs0, bs1)
        bdsem = (bd0, bd1)
        gsem = (gs0, gs1)
        dsem = (gd0, gd1)
        osem = (os0, os1)

        c = lax.axis_index("c")
        s = lax.axis_index("s")
        cpw = jnp.where(c == 0, cpw0, cpw1)
        nsuper = cpw // SUP
        base_chunk = jnp.where(c == 0, s * cpw0, NS * cpw0 + s * cpw1)
        base0 = base_chunk * C
        lane = lax.broadcasted_iota(jnp.int32, (L,), 0)

        def launch_batch(sup, hb):
            off = base0 + sup * (SUP * C)
            pltpu.async_copy(src_hbm.at[pl.ds(off, SUP * C)], sbatch[hb],
                             bssem[hb])
            pltpu.async_copy(dst_hbm.at[pl.ds(off, SUP * C)], dbatch[hb],
                             bdsem[hb])

        def wait_batch(hb):
            pltpu.make_async_copy(src_hbm.at[pl.ds(base0, SUP * C)],
                                  sbatch[hb], bssem[hb]).wait()
            pltpu.make_async_copy(dst_hbm.at[pl.ds(base0, SUP * C)],
                                  dbatch[hb], bdsem[hb]).wait()

        def stage(j, pos, hb, b):
            sl = pl.ds(pos * C, C)
            pltpu.async_copy(z_hbm.at[sbatch[hb].at[sl]], srows[b], gsem[b])
            pltpu.async_copy(z_hbm.at[dbatch[hb].at[sl]], drows[b], dsem[b])

        def compute(b):
            def group_body(g, carry2):
                def quad(q, res):
                    for i in range(4):
                        e = g * L + q * 4 + i
                        acc = (srows[b][e, pl.ds(0, L)]
                               * drows[b][e, pl.ds(0, L)])
                        for k8 in range(1, D // L):
                            a = srows[b][e, pl.ds(k8 * L, L)]
                            bb = drows[b][e, pl.ds(k8 * L, L)]
                            acc = acc + a * bb
                        res = jnp.where(lane == q * 4 + i, jnp.sum(acc), res)
                    return res

                res = lax.fori_loop(0, 4, quad, jnp.zeros((L,), jnp.float32))
                outv[b][pl.ds(g * L, L)] = res
                return carry2

            lax.fori_loop(0, C // L, group_body, 0)

        def process(j, b, stage_pos, stage_hb):
            pltpu.make_async_copy(z_hbm.at[sbatch[0].at[pl.ds(0, C)]],
                                  srows[b], gsem[b]).wait()
            pltpu.make_async_copy(z_hbm.at[dbatch[0].at[pl.ds(0, C)]],
                                  drows[b], dsem[b]).wait()

            @pl.when(j >= NBUF)
            def _():
                pltpu.make_async_copy(outv[b], out_hbm.at[pl.ds(base0, C)],
                                      osem[b]).wait()

            compute(b)
            pltpu.async_copy(outv[b], out_hbm.at[pl.ds(base0 + j * C, C)],
                             osem[b])

            @pl.when(j + NBUF < cpw)
            def _():
                stage(j + NBUF, stage_pos, stage_hb, b)

        launch_batch(0, 0)

        @pl.when(1 < nsuper)
        def _():
            launch_batch(1, 1)

        wait_batch(0)
        stage(0, 0, 0, 0)
        stage(1, 1, 0, 1)

        def super_pair(s2, carry):
            for h in range(2):
                sup = s2 * 2 + h
                j0 = sup * SUP
                process(j0 + 0, 0, 2, h)
                process(j0 + 1, 1, 3, h)

                @pl.when(sup + 1 < nsuper)
                def _():
                    wait_batch(h ^ 1)

                process(j0 + 2, 0, 0, h ^ 1)
                process(j0 + 3, 1, 1, h ^ 1)

                @pl.when(sup + 2 < nsuper)
                def _():
                    launch_batch(sup + 2, h)
            return carry

        lax.fori_loop(0, cpw // (2 * SUP), super_pair, 0)

        for b in range(NBUF):
            pltpu.make_async_copy(outv[b], out_hbm.at[pl.ds(base0, C)],
                                  osem[b]).wait()

    return kern(z, src, dst)


def kernel(features, graph, pos_edge, neg_edge):
    z = features[-1]
    edge = jnp.concatenate([pos_edge, neg_edge], axis=-1)
    e = edge.shape[1]
    unit = NS * C * 2 * SUP
    t = -(-e // unit) * 2 * SUP
    cpw1 = max(2 * SUP, int(t * FRAC1 / (2 * SUP)) * 2 * SUP)
    cpw0 = t - cpw1
    e_pad = NS * t * C
    src = jnp.pad(edge[0], (0, e_pad - e))
    dst = jnp.pad(edge[1], (0, e_pad - e))
    out = _decode(z, src, dst, cpw0, cpw1)
    return out[:e]


# speedup vs baseline: 1.1848x; 1.0041x over previous
import functools

import jax
import jax.numpy as jnp
from jax import lax
from jax.experimental import pallas as pl
from jax.experimental.pallas import tpu as pltpu
from jax.experimental.pallas import tpu_sc as plsc

D = 128
C = 128
NC = 2
NS = 16
L = 16
NBUF = 2
NIDX = 4
FRAC1 = 0.30


def _decode(z, src, dst, cpw0, cpw1):
    e_pad = src.shape[0]
    mesh = plsc.VectorSubcoreMesh(core_axis_name="c", subcore_axis_name="s")

    @functools.partial(
        pl.kernel,
        mesh=mesh,
        compiler_params=pltpu.CompilerParams(needs_layout_passes=False),
        out_type=jax.ShapeDtypeStruct((e_pad,), jnp.float32),
        scratch_types=(
            [pltpu.VMEM((C,), jnp.int32) for _ in range(2 * NIDX)]
            + [pltpu.VMEM((C, D), jnp.float32) for _ in range(2 * NBUF)]
            + [pltpu.VMEM((C,), jnp.float32) for _ in range(NBUF)]
            + [pltpu.SemaphoreType.DMA for _ in range(2 * NIDX + 3 * NBUF)]
        ),
    )
    def kern(z_hbm, src_hbm, dst_hbm, out_hbm,
             si0, si1, si2, si3, di0, di1, di2, di3,
             sr0, sr1, dr0, dr1, ov0, ov1,
             is0, is1, is2, is3, id0, id1, id2, id3,
             gs0, gs1, gd0, gd1, os0, os1):
        sidx = (si0, si1, si2, si3)
        didx = (di0, di1, di2, di3)
        srows = (sr0, sr1)
        drows = (dr0, dr1)
        outv = (ov0, ov1)
        issem = (is0, is1, is2, is3)
        idsem = (id0, id1, id2, id3)
        gsem = (gs0, gs1)
        dsem = (gd0, gd1)
        osem = (os0, os1)

        c = lax.axis_index("c")
        s = lax.axis_index("s")
        cpw = jnp.where(c == 0, cpw0, cpw1)
        base_chunk = jnp.where(c == 0, s * cpw0, NS * cpw0 + s * cpw1)
        base0 = base_chunk * C
        lane = lax.broadcasted_iota(jnp.int32, (L,), 0)

        def launch_idx(j, p):
            off = base0 + j * C
            pltpu.async_copy(src_hbm.at[pl.ds(off, C)], sidx[p], issem[p])
            pltpu.async_copy(dst_hbm.at[pl.ds(off, C)], didx[p], idsem[p])

        def wait_idx(p):
            pltpu.make_async_copy(src_hbm.at[pl.ds(base0, C)], sidx[p],
                                  issem[p]).wait()
            pltpu.make_async_copy(dst_hbm.at[pl.ds(base0, C)], didx[p],
                                  idsem[p]).wait()

        def stage(p, b):
            pltpu.async_copy(z_hbm.at[sidx[p]], srows[b], gsem[b])
            pltpu.async_copy(z_hbm.at[didx[p]], drows[b], dsem[b])

        def compute(b):
            def group_body(g, carry2):
                def quad(q, res):
                    for i in range(4):
                        e = g * L + q * 4 + i
                        acc = (srows[b][e, pl.ds(0, L)]
                               * drows[b][e, pl.ds(0, L)])
                        for k8 in range(1, D // L):
                            a = srows[b][e, pl.ds(k8 * L, L)]
                            bb = drows[b][e, pl.ds(k8 * L, L)]
                            acc = acc + a * bb
                        res = jnp.where(lane == q * 4 + i, jnp.sum(acc), res)
                    return res

                res = lax.fori_loop(0, 4, quad, jnp.zeros((L,), jnp.float32))
                outv[b][pl.ds(g * L, L)] = res
                return carry2

            lax.fori_loop(0, C // L, group_body, 0)

        def process(j, p):
            b = p % NBUF
            pltpu.make_async_copy(z_hbm.at[sidx[p]], srows[b], gsem[b]).wait()
            pltpu.make_async_copy(z_hbm.at[didx[p]], drows[b], dsem[b]).wait()

            @pl.when(j + NIDX < cpw)
            def _():
                launch_idx(j + NIDX, p)

            @pl.when(j >= NBUF)
            def _():
                pltpu.make_async_copy(outv[b], out_hbm.at[pl.ds(base0, C)],
                                      osem[b]).wait()

            compute(b)
            pltpu.async_copy(outv[b], out_hbm.at[pl.ds(base0 + j * C, C)],
                             osem[b])

            pnext = (p + NBUF) % NIDX

            @pl.when(j + NBUF < cpw)
            def _():
                wait_idx(pnext)
                stage(pnext, b)

        for jj in range(NIDX):
            launch_idx(jj, jj)
        wait_idx(0)
        stage(0, 0)
        wait_idx(1)
        stage(1, 1)

        def quad_body(i, carry):
            j0 = i * NIDX
            for pp in range(NIDX):
                process(j0 + pp, pp)
            return carry

        lax.fori_loop(0, cpw // NIDX, quad_body, 0)

        for b in range(NBUF):
            pltpu.make_async_copy(outv[b], out_hbm.at[pl.ds(base0, C)],
                                  osem[b]).wait()

    return kern(z, src, dst)


def kernel(features, graph, pos_edge, neg_edge):
    z = features[-1]
    edge = jnp.concatenate([pos_edge, neg_edge], axis=-1)
    e = edge.shape[1]
    unit = NS * C * NIDX
    t = -(-e // unit) * NIDX
    cpw1 = max(NIDX, int(t * FRAC1 / NIDX) * NIDX)
    cpw0 = t - cpw1
    e_pad = NS * t * C
    src = jnp.pad(edge[0], (0, e_pad - e))
    dst = jnp.pad(edge[1], (0, e_pad - e))
    out = _decode(z, src, dst, cpw0, cpw1)
    return out[:e]


# speedup vs baseline: 1.2053x; 1.0173x over previous
import functools

import jax
import jax.numpy as jnp
from jax import lax
from jax.experimental import pallas as pl
from jax.experimental.pallas import tpu as pltpu
from jax.experimental.pallas import tpu_sc as plsc

D = 128
C = 128
NC = 2
NS = 16
L = 16
NBUF = 2
NIDX = 4
FRAC1 = 0.18


def _decode(z, src, dst, cpw0, cpw1):
    e_pad = src.shape[0]
    mesh = plsc.VectorSubcoreMesh(core_axis_name="c", subcore_axis_name="s")

    @functools.partial(
        pl.kernel,
        mesh=mesh,
        compiler_params=pltpu.CompilerParams(needs_layout_passes=False),
        out_type=jax.ShapeDtypeStruct((e_pad,), jnp.float32),
        scratch_types=(
            [pltpu.VMEM((C,), jnp.int32) for _ in range(2 * NIDX)]
            + [pltpu.VMEM((C, D), jnp.float32) for _ in range(2 * NBUF)]
            + [pltpu.VMEM((C,), jnp.float32) for _ in range(NBUF)]
            + [pltpu.SemaphoreType.DMA for _ in range(2 * NIDX + 3 * NBUF)]
        ),
    )
    def kern(z_hbm, src_hbm, dst_hbm, out_hbm,
             si0, si1, si2, si3, di0, di1, di2, di3,
             sr0, sr1, dr0, dr1, ov0, ov1,
             is0, is1, is2, is3, id0, id1, id2, id3,
             gs0, gs1, gd0, gd1, os0, os1):
        sidx = (si0, si1, si2, si3)
        didx = (di0, di1, di2, di3)
        srows = (sr0, sr1)
        drows = (dr0, dr1)
        outv = (ov0, ov1)
        issem = (is0, is1, is2, is3)
        idsem = (id0, id1, id2, id3)
        gsem = (gs0, gs1)
        dsem = (gd0, gd1)
        osem = (os0, os1)

        c = lax.axis_index("c")
        s = lax.axis_index("s")
        cpw = jnp.where(c == 0, cpw0, cpw1)
        base_chunk = jnp.where(c == 0, s * cpw0, NS * cpw0 + s * cpw1)
        base0 = base_chunk * C
        lane = lax.broadcasted_iota(jnp.int32, (L,), 0)

        def launch_idx(j, p):
            off = base0 + j * C
            pltpu.async_copy(src_hbm.at[pl.ds(off, C)], sidx[p], issem[p])
            pltpu.async_copy(dst_hbm.at[pl.ds(off, C)], didx[p], idsem[p])

        def wait_idx(p):
            pltpu.make_async_copy(src_hbm.at[pl.ds(base0, C)], sidx[p],
                                  issem[p]).wait()
            pltpu.make_async_copy(dst_hbm.at[pl.ds(base0, C)], didx[p],
                                  idsem[p]).wait()

        def stage(p, b):
            pltpu.async_copy(z_hbm.at[sidx[p]], srows[b], gsem[b])
            pltpu.async_copy(z_hbm.at[didx[p]], drows[b], dsem[b])

        def compute(b):
            def group_body(g, carry2):
                def quad(q, res):
                    for i in range(4):
                        e = g * L + q * 4 + i
                        acc = (srows[b][e, pl.ds(0, L)]
                               * drows[b][e, pl.ds(0, L)])
                        for k8 in range(1, D // L):
                            a = srows[b][e, pl.ds(k8 * L, L)]
                            bb = drows[b][e, pl.ds(k8 * L, L)]
                            acc = acc + a * bb
                        res = jnp.where(lane == q * 4 + i, jnp.sum(acc), res)
                    return res

                res = lax.fori_loop(0, 4, quad, jnp.zeros((L,), jnp.float32))
                outv[b][pl.ds(g * L, L)] = res
                return carry2

            lax.fori_loop(0, C // L, group_body, 0)

        def process(j, p):
            b = p % NBUF
            pltpu.make_async_copy(z_hbm.at[sidx[p]], srows[b], gsem[b]).wait()
            pltpu.make_async_copy(z_hbm.at[didx[p]], drows[b], dsem[b]).wait()

            @pl.when(j + NIDX < cpw)
            def _():
                launch_idx(j + NIDX, p)

            @pl.when(j >= NBUF)
            def _():
                pltpu.make_async_copy(outv[b], out_hbm.at[pl.ds(base0, C)],
                                      osem[b]).wait()

            compute(b)
            pltpu.async_copy(outv[b], out_hbm.at[pl.ds(base0 + j * C, C)],
                             osem[b])

            pnext = (p + NBUF) % NIDX

            @pl.when(j + NBUF < cpw)
            def _():
                wait_idx(pnext)
                stage(pnext, b)

        for jj in range(NIDX):
            launch_idx(jj, jj)
        wait_idx(0)
        stage(0, 0)
        wait_idx(1)
        stage(1, 1)

        def quad_body(i, carry):
            j0 = i * NIDX
            for pp in range(NIDX):
                process(j0 + pp, pp)
            return carry

        lax.fori_loop(0, cpw // NIDX, quad_body, 0)

        for b in range(NBUF):
            pltpu.make_async_copy(outv[b], out_hbm.at[pl.ds(base0, C)],
                                  osem[b]).wait()

    return kern(z, src, dst)


def kernel(features, graph, pos_edge, neg_edge):
    z = features[-1]
    edge = jnp.concatenate([pos_edge, neg_edge], axis=-1)
    e = edge.shape[1]
    unit = NS * C * NIDX
    t = -(-e // unit) * NIDX
    cpw1 = max(NIDX, int(t * FRAC1 / NIDX) * NIDX)
    cpw0 = t - cpw1
    e_pad = NS * t * C
    src = jnp.pad(edge[0], (0, e_pad - e))
    dst = jnp.pad(edge[1], (0, e_pad - e))
    out = _decode(z, src, dst, cpw0, cpw1)
    return out[:e]


# speedup vs baseline: 1.7432x; 1.4463x over previous
import functools

import jax
import jax.numpy as jnp
from jax import lax
from jax.experimental import pallas as pl
from jax.experimental.pallas import tpu as pltpu
from jax.experimental.pallas import tpu_sc as plsc

D = 128
C = 128
NC = 2
NS = 16
L = 16
NBUF = 2
FRAC1 = 0.30


def _decode(z, src, dst, cpw0, cpw1):
    e_pad = src.shape[0]
    mesh = plsc.VectorSubcoreMesh(core_axis_name="c", subcore_axis_name="s")

    @functools.partial(
        pl.kernel,
        mesh=mesh,
        compiler_params=pltpu.CompilerParams(needs_layout_passes=False),
        out_type=jax.ShapeDtypeStruct((e_pad,), jnp.float32),
        scratch_types=(
            [pltpu.VMEM((C,), jnp.int32) for _ in range(2 * NBUF)]
            + [pltpu.VMEM((C, D), jnp.float32) for _ in range(2 * NBUF)]
            + [pltpu.VMEM((C,), jnp.float32) for _ in range(NBUF)]
            + [pltpu.SemaphoreType.DMA for _ in range(3 * NBUF)]
        ),
    )
    def kern(z_hbm, src_hbm, dst_hbm, out_hbm,
             sidx0, sidx1, didx0, didx1, sr0, sr1, dr0, dr1, ov0, ov1,
             gs0, gs1, gd0, gd1, os0, os1):
        sidx = (sidx0, sidx1)
        didx = (didx0, didx1)
        srows = (sr0, sr1)
        drows = (dr0, dr1)
        outv = (ov0, ov1)
        gsem = (gs0, gs1)
        dsem = (gd0, gd1)
        osem = (os0, os1)

        c = lax.axis_index("c")
        s = lax.axis_index("s")
        cpw = jnp.where(c == 0, cpw0, cpw1)
        base_chunk = jnp.where(c == 0, s * cpw0, NS * cpw0 + s * cpw1)
        base0 = base_chunk * C
        lane = lax.broadcasted_iota(jnp.int32, (L,), 0)

        def stage(j, b):
            off = base0 + j * C
            pltpu.sync_copy(src_hbm.at[pl.ds(off, C)], sidx[b])
            pltpu.sync_copy(dst_hbm.at[pl.ds(off, C)], didx[b])
            pltpu.async_copy(z_hbm.at[sidx[b]], srows[b], gsem[b])
            pltpu.async_copy(z_hbm.at[didx[b]], drows[b], dsem[b])

        def compute(b):
            def group_body(g, carry2):
                def quad(q, res):
                    for i in range(4):
                        e = g * L + q * 4 + i
                        acc = (srows[b][e, pl.ds(0, L)]
                               * drows[b][e, pl.ds(0, L)])
                        for k8 in range(1, D // L):
                            a = srows[b][e, pl.ds(k8 * L, L)]
                            bb = drows[b][e, pl.ds(k8 * L, L)]
                            acc = acc + a * bb
                        res = jnp.where(lane == q * 4 + i, jnp.sum(acc), res)
                    return res

                res = lax.fori_loop(0, 4, quad, jnp.zeros((L,), jnp.float32))
                outv[b][pl.ds(g * L, L)] = res
                return carry2

            lax.fori_loop(0, C // L, group_body, 0)

        for b in range(NBUF):
            stage(b, b)

        def loop_body(i, carry):
            for b in range(NBUF):
                j = i * NBUF + b
                pltpu.make_async_copy(z_hbm.at[sidx[b]], srows[b],
                                      gsem[b]).wait()
                pltpu.make_async_copy(z_hbm.at[didx[b]], drows[b],
                                      dsem[b]).wait()

                @pl.when(j >= NBUF)
                def _():
                    pltpu.make_async_copy(outv[b],
                                          out_hbm.at[pl.ds(base0, C)],
                                          osem[b]).wait()

                compute(b)
                off = base0 + j * C
                pltpu.async_copy(outv[b], out_hbm.at[pl.ds(off, C)], osem[b])

                nj = j + NBUF

                @pl.when(nj < cpw)
                def _():
                    stage(nj, b)
            return carry

        lax.fori_loop(0, cpw // NBUF, loop_body, 0)

        for b in range(NBUF):
            pltpu.make_async_copy(outv[b], out_hbm.at[pl.ds(base0, C)],
                                  osem[b]).wait()

    return kern(z, src, dst)


def kernel(features, graph, pos_edge, neg_edge):
    z = features[-1]
    edge = jnp.concatenate([pos_edge, neg_edge], axis=-1)
    e = edge.shape[1]
    unit = NS * C * NBUF
    t = -(-e // unit) * NBUF
    cpw1 = max(NBUF, int(t * FRAC1 / NBUF) * NBUF)
    cpw0 = t - cpw1
    e_pad = NS * t * C
    src = jnp.pad(edge[0], (0, e_pad - e))
    dst = jnp.pad(edge[1], (0, e_pad - e))
    out = _decode(z, src, dst, cpw0, cpw1)
    return out[:e]


# speedup vs baseline: 1.7667x; 1.0135x over previous
import functools

import jax
import jax.numpy as jnp
from jax import lax
from jax.experimental import pallas as pl
from jax.experimental.pallas import tpu as pltpu
from jax.experimental.pallas import tpu_sc as plsc

D = 128
C = 128
NC = 2
NS = 16
L = 16
NBUF = 3
FRAC1 = 0.30


def _decode(z, src, dst, cpw0, cpw1):
    e_pad = src.shape[0]
    mesh = plsc.VectorSubcoreMesh(core_axis_name="c", subcore_axis_name="s")

    @functools.partial(
        pl.kernel,
        mesh=mesh,
        compiler_params=pltpu.CompilerParams(needs_layout_passes=False),
        out_type=jax.ShapeDtypeStruct((e_pad,), jnp.float32),
        scratch_types=(
            [pltpu.VMEM((C,), jnp.int32) for _ in range(2 * NBUF)]
            + [pltpu.VMEM((C, D), jnp.float32) for _ in range(2 * NBUF)]
            + [pltpu.VMEM((C,), jnp.float32) for _ in range(NBUF)]
            + [pltpu.SemaphoreType.DMA for _ in range(3 * NBUF)]
        ),
    )
    def kern(z_hbm, src_hbm, dst_hbm, out_hbm,
             sidx0, sidx1, sidx2, didx0, didx1, didx2,
             sr0, sr1, sr2, dr0, dr1, dr2, ov0, ov1, ov2,
             gs0, gs1, gs2, gd0, gd1, gd2, os0, os1, os2):
        sidx = (sidx0, sidx1, sidx2)
        didx = (didx0, didx1, didx2)
        srows = (sr0, sr1, sr2)
        drows = (dr0, dr1, dr2)
        outv = (ov0, ov1, ov2)
        gsem = (gs0, gs1, gs2)
        dsem = (gd0, gd1, gd2)
        osem = (os0, os1, os2)

        c = lax.axis_index("c")
        s = lax.axis_index("s")
        cpw = jnp.where(c == 0, cpw0, cpw1)
        base_chunk = jnp.where(c == 0, s * cpw0, NS * cpw0 + s * cpw1)
        base0 = base_chunk * C
        lane = lax.broadcasted_iota(jnp.int32, (L,), 0)

        def stage(j, b):
            off = base0 + j * C
            pltpu.sync_copy(src_hbm.at[pl.ds(off, C)], sidx[b])
            pltpu.sync_copy(dst_hbm.at[pl.ds(off, C)], didx[b])
            pltpu.async_copy(z_hbm.at[sidx[b]], srows[b], gsem[b])
            pltpu.async_copy(z_hbm.at[didx[b]], drows[b], dsem[b])

        def compute(b):
            def group_body(g, carry2):
                def quad(q, res):
                    for i in range(4):
                        e = g * L + q * 4 + i
                        acc = (srows[b][e, pl.ds(0, L)]
                               * drows[b][e, pl.ds(0, L)])
                        for k8 in range(1, D // L):
                            a = srows[b][e, pl.ds(k8 * L, L)]
                            bb = drows[b][e, pl.ds(k8 * L, L)]
                            acc = acc + a * bb
                        res = jnp.where(lane == q * 4 + i, jnp.sum(acc), res)
                    return res

                res = lax.fori_loop(0, 4, quad, jnp.zeros((L,), jnp.float32))
                outv[b][pl.ds(g * L, L)] = res
                return carry2

            lax.fori_loop(0, C // L, group_body, 0)

        for b in range(NBUF):
            stage(b, b)

        def loop_body(i, carry):
            for b in range(NBUF):
                j = i * NBUF + b
                pltpu.make_async_copy(z_hbm.at[sidx[b]], srows[b],
                                      gsem[b]).wait()
                pltpu.make_async_copy(z_hbm.at[didx[b]], drows[b],
                                      dsem[b]).wait()

                @pl.when(j >= NBUF)
                def _():
                    pltpu.make_async_copy(outv[b],
                                          out_hbm.at[pl.ds(base0, C)],
                                          osem[b]).wait()

                compute(b)
                off = base0 + j * C
                pltpu.async_copy(outv[b], out_hbm.at[pl.ds(off, C)], osem[b])

                nj = j + NBUF

                @pl.when(nj < cpw)
                def _():
                    stage(nj, b)
            return carry

        lax.fori_loop(0, cpw // NBUF, loop_body, 0)

        for b in range(NBUF):
            pltpu.make_async_copy(outv[b], out_hbm.at[pl.ds(base0, C)],
                                  osem[b]).wait()

    return kern(z, src, dst)


def kernel(features, graph, pos_edge, neg_edge):
    z = features[-1]
    edge = jnp.concatenate([pos_edge, neg_edge], axis=-1)
    e = edge.shape[1]
    unit = NS * C * NBUF
    t = -(-e // unit) * NBUF
    cpw1 = max(NBUF, int(t * FRAC1 / NBUF) * NBUF)
    cpw0 = t - cpw1
    e_pad = NS * t * C
    src = jnp.pad(edge[0], (0, e_pad - e))
    dst = jnp.pad(edge[1], (0, e_pad - e))
    out = _decode(z, src, dst, cpw0, cpw1)
    return out[:e]


# speedup vs baseline: 1.8279x; 1.0346x over previous
import functools

import jax
import jax.numpy as jnp
from jax import lax
from jax.experimental import pallas as pl
from jax.experimental.pallas import tpu as pltpu
from jax.experimental.pallas import tpu_sc as plsc

D = 128
C = 128
NC = 2
NS = 16
L = 16
NBUF = 3
FRAC1 = 0.33


def _decode(z, src, dst, cpw0, cpw1):
    e_pad = src.shape[0]
    mesh = plsc.VectorSubcoreMesh(core_axis_name="c", subcore_axis_name="s")

    @functools.partial(
        pl.kernel,
        mesh=mesh,
        compiler_params=pltpu.CompilerParams(needs_layout_passes=False),
        out_type=jax.ShapeDtypeStruct((e_pad,), jnp.float32),
        scratch_types=(
            [pltpu.VMEM((C,), jnp.int32) for _ in range(2 * NBUF)]
            + [pltpu.VMEM((C, D), jnp.float32) for _ in range(2 * NBUF)]
            + [pltpu.VMEM((C,), jnp.float32) for _ in range(NBUF)]
            + [pltpu.SemaphoreType.DMA for _ in range(3 * NBUF)]
        ),
    )
    def kern(z_hbm, src_hbm, dst_hbm, out_hbm,
             sidx0, sidx1, sidx2, didx0, didx1, didx2,
             sr0, sr1, sr2, dr0, dr1, dr2, ov0, ov1, ov2,
             gs0, gs1, gs2, gd0, gd1, gd2, os0, os1, os2):
        sidx = (sidx0, sidx1, sidx2)
        didx = (didx0, didx1, didx2)
        srows = (sr0, sr1, sr2)
        drows = (dr0, dr1, dr2)
        outv = (ov0, ov1, ov2)
        gsem = (gs0, gs1, gs2)
        dsem = (gd0, gd1, gd2)
        osem = (os0, os1, os2)

        c = lax.axis_index("c")
        s = lax.axis_index("s")
        cpw = jnp.where(c == 0, cpw0, cpw1)
        base_chunk = jnp.where(c == 0, s * cpw0, NS * cpw0 + s * cpw1)
        base0 = base_chunk * C
        lane = lax.broadcasted_iota(jnp.int32, (L,), 0)

        def stage(j, b):
            off = base0 + j * C
            pltpu.sync_copy(src_hbm.at[pl.ds(off, C)], sidx[b])
            pltpu.sync_copy(dst_hbm.at[pl.ds(off, C)], didx[b])
            pltpu.async_copy(z_hbm.at[sidx[b]], srows[b], gsem[b])
            pltpu.async_copy(z_hbm.at[didx[b]], drows[b], dsem[b])

        def compute(b):
            def group_body(g, carry2):
                def quad(q, res):
                    for i in range(4):
                        e = g * L + q * 4 + i
                        acc = (srows[b][e, pl.ds(0, L)]
                               * drows[b][e, pl.ds(0, L)])
                        for k8 in range(1, D // L):
                            a = srows[b][e, pl.ds(k8 * L, L)]
                            bb = drows[b][e, pl.ds(k8 * L, L)]
                            acc = acc + a * bb
                        res = jnp.where(lane == q * 4 + i, jnp.sum(acc), res)
                    return res

                res = lax.fori_loop(0, 4, quad, jnp.zeros((L,), jnp.float32))
                outv[b][pl.ds(g * L, L)] = res
                return carry2

            lax.fori_loop(0, C // L, group_body, 0)

        for b in range(NBUF):
            stage(b, b)

        def loop_body(i, carry):
            for b in range(NBUF):
                j = i * NBUF + b
                pltpu.make_async_copy(z_hbm.at[sidx[b]], srows[b],
                                      gsem[b]).wait()
                pltpu.make_async_copy(z_hbm.at[didx[b]], drows[b],
                                      dsem[b]).wait()

                @pl.when(j >= NBUF)
                def _():
                    pltpu.make_async_copy(outv[b],
                                          out_hbm.at[pl.ds(base0, C)],
                                          osem[b]).wait()

                compute(b)
                off = base0 + j * C
                pltpu.async_copy(outv[b], out_hbm.at[pl.ds(off, C)], osem[b])

                nj = j + NBUF

                @pl.when(nj < cpw)
                def _():
                    stage(nj, b)
            return carry

        lax.fori_loop(0, cpw // NBUF, loop_body, 0)

        for b in range(NBUF):
            pltpu.make_async_copy(outv[b], out_hbm.at[pl.ds(base0, C)],
                                  osem[b]).wait()

    return kern(z, src, dst)


def kernel(features, graph, pos_edge, neg_edge):
    z = features[-1]
    edge = jnp.concatenate([pos_edge, neg_edge], axis=-1)
    e = edge.shape[1]
    unit = NS * C * NBUF
    t = -(-e // unit) * NBUF
    cpw1 = max(NBUF, int(t * FRAC1 / NBUF) * NBUF)
    cpw0 = t - cpw1
    e_pad = NS * t * C
    src = jnp.pad(edge[0], (0, e_pad - e))
    dst = jnp.pad(edge[1], (0, e_pad - e))
    out = _decode(z, src, dst, cpw0, cpw1)
    return out[:e]
